# baseline (speedup 1.0000x reference)
import functools
import jax
import jax.numpy as jnp
import numpy as np
from jax.experimental import pallas as pl
from jax.experimental.compute_on import compute_on


@compute_on("tpu_sparsecore")
@jax.jit
def _seg_reduce(m2, mpm, dst):
    s2 = jax.ops.segment_sum(m2, dst, 10000, indices_are_sorted=True)
    mxn = jax.ops.segment_max(mpm, dst, 10000, indices_are_sorted=True)
    return s2, mxn

L = 2
N = 10000
E = 320000
F = 128
NAF = 9
NBF = 3
_DEG = np.concatenate([np.zeros(32), np.array([10000.0])])
_BINS = np.arange(33).astype(np.float64)
_AVG_LOG = float((np.log(_BINS + 1.0) * _DEG).sum() / _DEG.sum())


def _matmul_kern(h_ref, w_ref, b_ref, o_ref):
    o_ref[...] = (
        jnp.dot(h_ref[...], w_ref[...], preferred_element_type=jnp.float32)
        + b_ref[...]
    )


def _matmul(h, w, b, block_rows=1000):
    n, k = h.shape
    k2, fo = w.shape
    grid = (n // block_rows,)
    return pl.pallas_call(
        _matmul_kern,
        grid=grid,
        in_specs=[
            pl.BlockSpec((block_rows, k), lambda i: (i, 0)),
            pl.BlockSpec((k, fo), lambda i: (0, 0)),
            pl.BlockSpec((1, fo), lambda i: (0, 0)),
        ],
        out_specs=pl.BlockSpec((block_rows, fo), lambda i: (i, 0)),
        out_shape=jax.ShapeDtypeStruct((n, fo), jnp.float32),
    )(h, w, b.reshape(1, -1))


def kernel(x, edge_index, edge_attr, atom_emb, bond_emb, pre_W, pre_b, post_W, post_b, lin_W, lin_b, bn_gamma, bn_beta):
    h = jnp.zeros((N, F), jnp.float32)
    for f in range(NAF):
        h = h + atom_emb[f][x[:, f]]
    src, dst = edge_index[0], edge_index[1]
    dst, perm = jax.lax.sort_key_val(dst, jnp.arange(E, dtype=jnp.int32))
    src = src[perm]
    ones = jnp.ones((E,), jnp.float32)
    cnt = jax.ops.segment_sum(ones, dst, N, indices_are_sorted=True)
    deg = cnt[:, None]
    logd = jnp.log(deg + 1.0)
    code = ((edge_attr[:, 0] * 8 + edge_attr[:, 1]) * 8 + edge_attr[:, 2])[perm]
    for l in range(L):
        W_i = pre_W[l, :F]
        W_j = pre_W[l, F : 2 * F]
        W_e = pre_W[l, 2 * F :]
        AB = _matmul(
            h,
            jnp.concatenate([W_i, W_j], axis=1),
            jnp.concatenate([pre_b[l], jnp.zeros((F,), jnp.float32)]),
        )
        A, B = AB[:, :F], AB[:, F:]
        T = (
            bond_emb[l, 0] @ W_e
        )[:, None, None, :] + (bond_emb[l, 1] @ W_e)[None, :, None, :] + (
            bond_emb[l, 2] @ W_e
        )[None, None, :, :]
        T = T.reshape(512, F)
        A2 = jnp.concatenate([A, -A], axis=1)
        B2 = jnp.concatenate([B, -B], axis=1)
        T2 = jnp.concatenate([T, -T], axis=1)
        mpm = A2[dst] + B2[src] + T2[code]
        m = mpm[:, :F]
        s2, mxn = _seg_reduce(jnp.concatenate([m, m * m], axis=1), mpm, dst)
        s, sq = s2[:, :F], s2[:, F:]
        mx, mn = mxn[:, :F], -mxn[:, F:]
        mean = s / jnp.maximum(deg, 1.0)
        mx = jnp.where(deg > 0, mx, 0.0)
        mn = jnp.where(deg > 0, mn, 0.0)
        msq = sq / jnp.maximum(deg, 1.0)
        std = jnp.sqrt(jax.nn.relu(msq - mean * mean) + 1e-5)
        agg = jnp.concatenate([mean, mn, mx, std], axis=-1)
        amp = agg * (logd / _AVG_LOG)
        att = agg * (_AVG_LOG / jnp.maximum(logd, 1e-6))
        scaled = jnp.concatenate([agg, amp, att], axis=-1)
        hs = jnp.concatenate([h, scaled], axis=-1)
        out = _matmul(hs, post_W[l] @ lin_W[l], post_b[l] @ lin_W[l] + lin_b[l])
        out = out / jnp.sqrt(1.0 + 1e-5) * bn_gamma[l] + bn_beta[l]
        h = jax.nn.relu(out) + h
    return h


# speedup vs baseline: 1.0754x; 1.0754x over previous
import functools
import jax
import jax.numpy as jnp
import numpy as np
from jax.experimental import pallas as pl
from jax.experimental.compute_on import compute_on


@compute_on("tpu_sparsecore")
@jax.jit
def _seg_reduce(m2, m, dst):
    s2 = jax.ops.segment_sum(m2, dst, 10000, indices_are_sorted=True)
    mx = jax.ops.segment_max(m, dst, 10000, indices_are_sorted=True)
    mn = jax.ops.segment_min(m, dst, 10000, indices_are_sorted=True)
    return s2, mx, mn

L = 2
N = 10000
E = 320000
F = 128
NAF = 9
NBF = 3
_DEG = np.concatenate([np.zeros(32), np.array([10000.0])])
_BINS = np.arange(33).astype(np.float64)
_AVG_LOG = float((np.log(_BINS + 1.0) * _DEG).sum() / _DEG.sum())


def _matmul_kern(h_ref, w_ref, b_ref, o_ref):
    o_ref[...] = (
        jnp.dot(h_ref[...], w_ref[...], preferred_element_type=jnp.float32)
        + b_ref[...]
    )


def _matmul(h, w, b, block_rows=1000):
    n, k = h.shape
    k2, fo = w.shape
    grid = (n // block_rows,)
    return pl.pallas_call(
        _matmul_kern,
        grid=grid,
        in_specs=[
            pl.BlockSpec((block_rows, k), lambda i: (i, 0)),
            pl.BlockSpec((k, fo), lambda i: (0, 0)),
            pl.BlockSpec((1, fo), lambda i: (0, 0)),
        ],
        out_specs=pl.BlockSpec((block_rows, fo), lambda i: (i, 0)),
        out_shape=jax.ShapeDtypeStruct((n, fo), jnp.float32),
    )(h, w, b.reshape(1, -1))


def kernel(x, edge_index, edge_attr, atom_emb, bond_emb, pre_W, pre_b, post_W, post_b, lin_W, lin_b, bn_gamma, bn_beta):
    h = jnp.zeros((N, F), jnp.float32)
    for f in range(NAF):
        h = h + atom_emb[f][x[:, f]]
    src, dst = edge_index[0], edge_index[1]
    dst, perm = jax.lax.sort_key_val(dst, jnp.arange(E, dtype=jnp.int32))
    src = src[perm]
    ones = jnp.ones((E,), jnp.float32)
    cnt = jax.ops.segment_sum(ones, dst, N, indices_are_sorted=True)
    deg = cnt[:, None]
    logd = jnp.log(deg + 1.0)
    code = ((edge_attr[:, 0] * 8 + edge_attr[:, 1]) * 8 + edge_attr[:, 2])[perm]
    for l in range(L):
        W_i = pre_W[l, :F]
        W_j = pre_W[l, F : 2 * F]
        W_e = pre_W[l, 2 * F :]
        AB = _matmul(
            h,
            jnp.concatenate([W_i, W_j], axis=1),
            jnp.concatenate([pre_b[l], jnp.zeros((F,), jnp.float32)]),
        )
        A, B = AB[:, :F], AB[:, F:]
        T = (
            bond_emb[l, 0] @ W_e
        )[:, None, None, :] + (bond_emb[l, 1] @ W_e)[None, :, None, :] + (
            bond_emb[l, 2] @ W_e
        )[None, None, :, :]
        T = T.reshape(512, F)
        m = A[dst] + B[src] + T[code]
        s2, mx, mn = _seg_reduce(jnp.concatenate([m, m * m], axis=1), m, dst)
        s, sq = s2[:, :F], s2[:, F:]
        mean = s / jnp.maximum(deg, 1.0)
        mx = jnp.where(deg > 0, mx, 0.0)
        mn = jnp.where(deg > 0, mn, 0.0)
        msq = sq / jnp.maximum(deg, 1.0)
        std = jnp.sqrt(jax.nn.relu(msq - mean * mean) + 1e-5)
        agg = jnp.concatenate([mean, mn, mx, std], axis=-1)
        amp = agg * (logd / _AVG_LOG)
        att = agg * (_AVG_LOG / jnp.maximum(logd, 1e-6))
        scaled = jnp.concatenate([agg, amp, att], axis=-1)
        hs = jnp.concatenate([h, scaled], axis=-1)
        out = _matmul(hs, post_W[l] @ lin_W[l], post_b[l] @ lin_W[l] + lin_b[l])
        out = out / jnp.sqrt(1.0 + 1e-5) * bn_gamma[l] + bn_beta[l]
        h = jax.nn.relu(out) + h
    return h


# speedup vs baseline: 1.0806x; 1.0049x over previous
import functools
import jax
import jax.numpy as jnp
import numpy as np
from jax.experimental import pallas as pl
from jax.experimental.compute_on import compute_on


@compute_on("tpu_sparsecore")
@jax.jit
def _seg_reduce(m2, m, dst):
    s2 = jax.ops.segment_sum(m2, dst, 10000, indices_are_sorted=True)
    mx = jax.ops.segment_max(m, dst, 10000, indices_are_sorted=True)
    mn = jax.ops.segment_min(m, dst, 10000, indices_are_sorted=True)
    return s2, mx, mn

L = 2
N = 10000
E = 320000
F = 128
NAF = 9
NBF = 3
_DEG = np.concatenate([np.zeros(32), np.array([10000.0])])
_BINS = np.arange(33).astype(np.float64)
_AVG_LOG = float((np.log(_BINS + 1.0) * _DEG).sum() / _DEG.sum())


def _matmul_kern(h_ref, w_ref, b_ref, o_ref):
    o_ref[...] = (
        jnp.dot(h_ref[...], w_ref[...], preferred_element_type=jnp.float32)
        + b_ref[...]
    )


def _matmul(h, w, b, block_rows=1000):
    n, k = h.shape
    k2, fo = w.shape
    grid = (n // block_rows,)
    return pl.pallas_call(
        _matmul_kern,
        grid=grid,
        in_specs=[
            pl.BlockSpec((block_rows, k), lambda i: (i, 0)),
            pl.BlockSpec((k, fo), lambda i: (0, 0)),
            pl.BlockSpec((1, fo), lambda i: (0, 0)),
        ],
        out_specs=pl.BlockSpec((block_rows, fo), lambda i: (i, 0)),
        out_shape=jax.ShapeDtypeStruct((n, fo), jnp.float32),
    )(h, w, b.reshape(1, -1))


def _fin_kern(h_ref, s2_ref, mn_ref, mx_ref, cnt_ref, pw_ref, pb_ref, g_ref,
              bt_ref, o_ref):
    h = h_ref[...]
    cnt = cnt_ref[...]
    pos = cnt > 0.0
    d = jnp.maximum(cnt, 1.0)
    mean = s2_ref[:, :F] / d
    msq = s2_ref[:, F:] / d
    std = jnp.sqrt(jax.nn.relu(msq - mean * mean) + 1e-5)
    mn = jnp.where(pos, mn_ref[...], 0.0)
    mx = jnp.where(pos, mx_ref[...], 0.0)
    agg = jnp.concatenate([mean, mn, mx, std], axis=-1)
    logd = jnp.log(cnt + 1.0)
    amp = agg * (logd / _AVG_LOG)
    att = agg * (_AVG_LOG / jnp.maximum(logd, 1e-6))
    hs = jnp.concatenate([h, agg, amp, att], axis=-1)
    out = jnp.dot(hs, pw_ref[...], preferred_element_type=jnp.float32) + pb_ref[...]
    out = out * (1.0 / np.sqrt(1.0 + 1e-5)) * g_ref[...] + bt_ref[...]
    o_ref[...] = jax.nn.relu(out) + h


def _fin_call(h, s2, mn, mx, cnt, pw, pb, gamma, beta):
    row = lambda i: (i, 0)
    zero = lambda i: (0, 0)
    return pl.pallas_call(
        _fin_kern,
        grid=(10,),
        in_specs=[
            pl.BlockSpec((1000, F), row),
            pl.BlockSpec((1000, 2 * F), row),
            pl.BlockSpec((1000, F), row),
            pl.BlockSpec((1000, F), row),
            pl.BlockSpec((1000, 1), row),
            pl.BlockSpec((13 * F, F), zero),
            pl.BlockSpec((1, F), zero),
            pl.BlockSpec((1, F), zero),
            pl.BlockSpec((1, F), zero),
        ],
        out_specs=pl.BlockSpec((1000, F), row),
        out_shape=jax.ShapeDtypeStruct((N, F), jnp.float32),
    )(h, s2, mn, mx, cnt.reshape(N, 1), pw, pb.reshape(1, F),
      gamma.reshape(1, F), beta.reshape(1, F))


def kernel(x, edge_index, edge_attr, atom_emb, bond_emb, pre_W, pre_b, post_W, post_b, lin_W, lin_b, bn_gamma, bn_beta):
    h = jnp.zeros((N, F), jnp.float32)
    for f in range(NAF):
        h = h + atom_emb[f][x[:, f]]
    src, dst = edge_index[0], edge_index[1]
    dst, perm = jax.lax.sort_key_val(dst, jnp.arange(E, dtype=jnp.int32))
    src = src[perm]
    ones = jnp.ones((E,), jnp.float32)
    cnt = jax.ops.segment_sum(ones, dst, N, indices_are_sorted=True)
    code = ((edge_attr[:, 0] * 8 + edge_attr[:, 1]) * 8 + edge_attr[:, 2])[perm]
    for l in range(L):
        W_i = pre_W[l, :F]
        W_j = pre_W[l, F : 2 * F]
        W_e = pre_W[l, 2 * F :]
        AB = _matmul(
            h,
            jnp.concatenate([W_i, W_j], axis=1),
            jnp.concatenate([pre_b[l], jnp.zeros((F,), jnp.float32)]),
        )
        A, B = AB[:, :F], AB[:, F:]
        T = (
            bond_emb[l, 0] @ W_e
        )[:, None, None, :] + (bond_emb[l, 1] @ W_e)[None, :, None, :] + (
            bond_emb[l, 2] @ W_e
        )[None, None, :, :]
        T = T.reshape(512, F)
        m = A[dst] + B[src] + T[code]
        s2, mx, mn = _seg_reduce(jnp.concatenate([m, m * m], axis=1), m, dst)
        h = _fin_call(h, s2, mn, mx, cnt, post_W[l] @ lin_W[l],
                      post_b[l] @ lin_W[l] + lin_b[l], bn_gamma[l], bn_beta[l])
    return h
